# baseline (device time: 138813 ns/iter reference)
import jax
import jax.numpy as jnp
from jax import lax
from jax.experimental import pallas as pl
from jax.experimental.pallas import tpu as pltpu

N_DEV = 4
B, S, D = 1, 1024, 2048
DC = 512
H, DH, DR = 16, 128, 32
SCALE = (DH + DR) ** -0.5


def _ring_kv_kernel(x2d, Wdkv, Wuk, Wuv, Wkr, Wqr):
    dc_sh = Wdkv.shape[1]

    half = Wdkv.shape[1] // 2

    def body(x_ref, wdkv_ref, wuk_ref, wuv_ref, wkr_ref, wqr_ref,
             xbf_ref, k_ref, v_ref, kr_ref, qrt_ref,
             cbufR, kbufR, vbufR, cbufL, kbufL, vbufL,
             send_semsR, recv_semsR, send_semsL, recv_semsL):
        my = lax.axis_index("i")
        left = lax.rem(my + N_DEV - 1, N_DEV)
        right = lax.rem(my + 1, N_DEV)

        bar = pltpu.get_barrier_semaphore()
        pl.semaphore_signal(bar, inc=1, device_id=(left,),
                            device_id_type=pl.DeviceIdType.MESH)
        pl.semaphore_signal(bar, inc=1, device_id=(right,),
                            device_id_type=pl.DeviceIdType.MESH)
        pl.semaphore_wait(bar, 2)

        xbf = x_ref[...].astype(jnp.bfloat16)
        xbf_ref[...] = xbf
        wukbf = wuk_ref[...].astype(jnp.bfloat16)
        wuvbf = wuv_ref[...].astype(jnp.bfloat16)
        cbf = jnp.dot(xbf, wdkv_ref[...].astype(jnp.bfloat16),
                      preferred_element_type=jnp.float32).astype(jnp.bfloat16)
        cbufR[0] = cbf[:, :half]
        cbufL[0] = cbf[:, half:]
        kbufR[0] = wukbf[:half, :]
        kbufL[0] = wukbf[half:, :]
        vbufR[0] = wuvbf[:half, :]
        vbufL[0] = wuvbf[half:, :]

        def start_hop(h):
            rdmas = []
            for bufs, ssems, rsems, tgt in (
                ((cbufR, kbufR, vbufR), send_semsR, recv_semsR, right),
                ((cbufL, kbufL, vbufL), send_semsL, recv_semsL, left),
            ):
                for bidx, buf in enumerate(bufs):
                    rdma = pltpu.make_async_remote_copy(
                        src_ref=buf.at[h],
                        dst_ref=buf.at[h + 1],
                        send_sem=ssems.at[h, bidx],
                        recv_sem=rsems.at[h, bidx],
                        device_id=(tgt,),
                        device_id_type=pl.DeviceIdType.MESH,
                    )
                    rdma.start()
                    rdmas.append(rdma)
            return rdmas

        hops = [start_hop(0)]

        k_ref[...] = jnp.dot(cbf, wukbf, preferred_element_type=jnp.float32)
        v_ref[...] = jnp.dot(cbf, wuvbf, preferred_element_type=jnp.float32)
        kr_ref[...] = jnp.dot(
            xbf, wkr_ref[...].astype(jnp.bfloat16),
            preferred_element_type=jnp.float32).astype(jnp.bfloat16)
        qrt_ref[...] = (lax.dot_general(
            wqr_ref[...].astype(jnp.bfloat16), xbf,
            (((0,), (1,)), ((), ())),
            preferred_element_type=jnp.float32) * SCALE).astype(jnp.bfloat16)

        for h in range(N_DEV - 1):
            for rdma in hops[h]:
                rdma.wait_recv()
            if h < N_DEV - 2:
                hops.append(start_hop(h + 1))
            k_ref[...] += jnp.dot(cbufR[h + 1], kbufR[h + 1],
                                  preferred_element_type=jnp.float32)
            k_ref[...] += jnp.dot(cbufL[h + 1], kbufL[h + 1],
                                  preferred_element_type=jnp.float32)
            v_ref[...] += jnp.dot(cbufR[h + 1], vbufR[h + 1],
                                  preferred_element_type=jnp.float32)
            v_ref[...] += jnp.dot(cbufL[h + 1], vbufL[h + 1],
                                  preferred_element_type=jnp.float32)

        for hop in hops:
            for rdma in hop:
                rdma.wait_send()

    return pl.pallas_call(
        body,
        out_shape=[
            jax.ShapeDtypeStruct((S, D), jnp.bfloat16),
            jax.ShapeDtypeStruct((S, D), jnp.float32),
            jax.ShapeDtypeStruct((S, D), jnp.float32),
            jax.ShapeDtypeStruct((S, DR), jnp.bfloat16),
            jax.ShapeDtypeStruct((DC, S), jnp.bfloat16),
        ],
        in_specs=[
            pl.BlockSpec((S, D), lambda: (0, 0)),
            pl.BlockSpec((D, dc_sh), lambda: (0, 0)),
            pl.BlockSpec((dc_sh, D), lambda: (0, 0)),
            pl.BlockSpec((dc_sh, D), lambda: (0, 0)),
            pl.BlockSpec((D, DR), lambda: (0, 0)),
            pl.BlockSpec((D, DC), lambda: (0, 0)),
        ],
        out_specs=[
            pl.BlockSpec((S, D), lambda: (0, 0)),
            pl.BlockSpec((S, D), lambda: (0, 0)),
            pl.BlockSpec((S, D), lambda: (0, 0)),
            pl.BlockSpec((S, DR), lambda: (0, 0)),
            pl.BlockSpec((DC, S), lambda: (0, 0)),
        ],
        scratch_shapes=[
            pltpu.VMEM((N_DEV, S, half), jnp.bfloat16),
            pltpu.VMEM((N_DEV, half, D), jnp.bfloat16),
            pltpu.VMEM((N_DEV, half, D), jnp.bfloat16),
            pltpu.VMEM((N_DEV, S, half), jnp.bfloat16),
            pltpu.VMEM((N_DEV, half, D), jnp.bfloat16),
            pltpu.VMEM((N_DEV, half, D), jnp.bfloat16),
            pltpu.SemaphoreType.DMA((N_DEV - 1, 3)),
            pltpu.SemaphoreType.DMA((N_DEV - 1, 3)),
            pltpu.SemaphoreType.DMA((N_DEV - 1, 3)),
            pltpu.SemaphoreType.DMA((N_DEV - 1, 3)),
        ],
        compiler_params=pltpu.CompilerParams(collective_id=0),
    )(x2d, Wdkv, Wuk, Wuv, Wkr, Wqr)


def _attention_kernel(xbf, K, V, Kr, QrT, Wq):

    def body(xbf_ref, k_ref, v_ref, kr_ref, qrt_ref, wq_ref, out_ref):
        xb = xbf_ref[...]
        q = (jnp.dot(xb, wq_ref[...].astype(jnp.bfloat16),
                     preferred_element_type=jnp.float32)
             * SCALE).astype(jnp.bfloat16)
        s = lax.dot_general(q, k_ref[...].astype(jnp.bfloat16),
                            (((1,), (1,)), ((), ())),
                            preferred_element_type=jnp.float32)
        s += lax.dot_general(qrt_ref[...], kr_ref[...],
                             (((0,), (1,)), ((), ())),
                             preferred_element_type=jnp.float32)
        p = jnp.exp(s).astype(jnp.bfloat16)
        v_aug = jnp.concatenate(
            [v_ref[...].astype(jnp.bfloat16),
             jnp.ones((S, DH), jnp.bfloat16)], axis=1)
        ov = jnp.dot(p, v_aug, preferred_element_type=jnp.float32)
        out_ref[...] = (ov[:, :DH] / ov[:, DH:DH + 1]).astype(jnp.bfloat16)

    return pl.pallas_call(
        body,
        grid=(H,),
        in_specs=[
            pl.BlockSpec((S, D), lambda h: (0, 0)),
            pl.BlockSpec((S, DH), lambda h: (0, h)),
            pl.BlockSpec((S, DH), lambda h: (0, h)),
            pl.BlockSpec((S, DR), lambda h: (0, 0)),
            pl.BlockSpec((DR, S), lambda h: (h, 0)),
            pl.BlockSpec((D, DH), lambda h: (0, h)),
        ],
        out_specs=pl.BlockSpec((S, DH), lambda h: (0, h)),
        out_shape=jax.ShapeDtypeStruct((S, D), jnp.bfloat16),
        compiler_params=pltpu.CompilerParams(
            dimension_semantics=("arbitrary",)),
    )(xbf, K, V, Kr, QrT, Wq)


_NJ = 8
_DJ = D // _NJ


def _out_proj_kernel(O, Wo):

    def body(o_ref, wo_ref, out_ref):
        out_ref[...] = jnp.dot(o_ref[...], wo_ref[...].astype(jnp.bfloat16),
                               preferred_element_type=jnp.float32)

    return pl.pallas_call(
        body,
        grid=(_NJ,),
        in_specs=[
            pl.BlockSpec((S, D), lambda j: (0, 0)),
            pl.BlockSpec((D, _DJ), lambda j: (0, j)),
        ],
        out_specs=pl.BlockSpec((S, _DJ), lambda j: (0, j)),
        out_shape=jax.ShapeDtypeStruct((S, D), jnp.float32),
        compiler_params=pltpu.CompilerParams(
            dimension_semantics=("arbitrary",)),
    )(O, Wo)


def kernel(x, Wdkv, Wuk, Wuv, Wq, Wqr, Wkr, Wo):
    x2d = x.reshape(S, D)
    xbf, K, V, Kr, QrT = _ring_kv_kernel(x2d, Wdkv, Wuk, Wuv, Wkr, Wqr)
    O = _attention_kernel(xbf, K, V, Kr, QrT, Wq)
    out = _out_proj_kernel(O, Wo)
    return out.reshape(B, S, D)


# device time: 118833 ns/iter; 1.1681x vs baseline; 1.1681x over previous
import jax
import jax.numpy as jnp
from jax import lax
from jax.experimental import pallas as pl
from jax.experimental.pallas import tpu as pltpu

N_DEV = 4
B, S, D = 1, 1024, 2048
DC = 512
H, DH, DR = 16, 128, 32
SCALE = (DH + DR) ** -0.5


def _ring_kv_kernel(x2d, Wdkv, Wuk, Wuv, Wkr, Wqr):
    dc_sh = Wdkv.shape[1]

    half = Wdkv.shape[1] // 2

    def body(x_ref, wdkv_ref, wuk_ref, wuv_ref, wkr_ref, wqr_ref,
             xbf_ref, k_ref, v_ref, kr_ref, qrt_ref,
             cbufR, kbufR, vbufR, cbufL, kbufL, vbufL,
             send_semsR, recv_semsR, send_semsL, recv_semsL):
        my = lax.axis_index("i")
        left = lax.rem(my + N_DEV - 1, N_DEV)
        right = lax.rem(my + 1, N_DEV)

        bar = pltpu.get_barrier_semaphore()
        pl.semaphore_signal(bar, inc=1, device_id=(left,),
                            device_id_type=pl.DeviceIdType.MESH)
        pl.semaphore_signal(bar, inc=1, device_id=(right,),
                            device_id_type=pl.DeviceIdType.MESH)
        pl.semaphore_wait(bar, 2)

        xbf = x_ref[...].astype(jnp.bfloat16)
        xbf_ref[...] = xbf
        wukbf = wuk_ref[...].astype(jnp.bfloat16)
        wuvbf = wuv_ref[...].astype(jnp.bfloat16)
        cbf = jnp.dot(xbf, wdkv_ref[...].astype(jnp.bfloat16),
                      preferred_element_type=jnp.float32).astype(jnp.bfloat16)
        cbufR[0] = cbf[:, :half]
        cbufL[0] = cbf[:, half:]
        kbufR[0] = wukbf[:half, :]
        kbufL[0] = wukbf[half:, :]
        vbufR[0] = wuvbf[:half, :]
        vbufL[0] = wuvbf[half:, :]

        def start_hop(h):
            rdmas = []
            for bufs, ssems, rsems, tgt in (
                ((cbufR, kbufR, vbufR), send_semsR, recv_semsR, right),
                ((cbufL, kbufL, vbufL), send_semsL, recv_semsL, left),
            ):
                for bidx, buf in enumerate(bufs):
                    rdma = pltpu.make_async_remote_copy(
                        src_ref=buf.at[h],
                        dst_ref=buf.at[h + 1],
                        send_sem=ssems.at[h, bidx],
                        recv_sem=rsems.at[h, bidx],
                        device_id=(tgt,),
                        device_id_type=pl.DeviceIdType.MESH,
                    )
                    rdma.start()
                    rdmas.append(rdma)
            return rdmas

        hops = [start_hop(0)]

        k_ref[...] = jnp.dot(cbf, wukbf, preferred_element_type=jnp.float32)
        v_ref[...] = jnp.dot(cbf, wuvbf, preferred_element_type=jnp.float32)
        kr_ref[...] = jnp.dot(
            xbf, wkr_ref[...].astype(jnp.bfloat16),
            preferred_element_type=jnp.float32).astype(jnp.bfloat16)
        qrt_ref[...] = (lax.dot_general(
            wqr_ref[...].astype(jnp.bfloat16), xbf,
            (((0,), (1,)), ((), ())),
            preferred_element_type=jnp.float32) * SCALE).astype(jnp.bfloat16)

        for h in range(N_DEV - 1):
            for rdma in hops[h]:
                rdma.wait_recv()
            if h < N_DEV - 2:
                hops.append(start_hop(h + 1))
            k_ref[...] += jnp.dot(cbufR[h + 1], kbufR[h + 1],
                                  preferred_element_type=jnp.float32)
            k_ref[...] += jnp.dot(cbufL[h + 1], kbufL[h + 1],
                                  preferred_element_type=jnp.float32)
            v_ref[...] += jnp.dot(cbufR[h + 1], vbufR[h + 1],
                                  preferred_element_type=jnp.float32)
            v_ref[...] += jnp.dot(cbufL[h + 1], vbufL[h + 1],
                                  preferred_element_type=jnp.float32)

        for hop in hops:
            for rdma in hop:
                rdma.wait_send()

    return pl.pallas_call(
        body,
        out_shape=[
            jax.ShapeDtypeStruct((S, D), jnp.bfloat16),
            jax.ShapeDtypeStruct((S, D), jnp.float32),
            jax.ShapeDtypeStruct((S, D), jnp.float32),
            jax.ShapeDtypeStruct((S, DR), jnp.bfloat16),
            jax.ShapeDtypeStruct((DC, S), jnp.bfloat16),
        ],
        in_specs=[
            pl.BlockSpec((S, D), lambda: (0, 0)),
            pl.BlockSpec((D, dc_sh), lambda: (0, 0)),
            pl.BlockSpec((dc_sh, D), lambda: (0, 0)),
            pl.BlockSpec((dc_sh, D), lambda: (0, 0)),
            pl.BlockSpec((D, DR), lambda: (0, 0)),
            pl.BlockSpec((D, DC), lambda: (0, 0)),
        ],
        out_specs=[
            pl.BlockSpec((S, D), lambda: (0, 0)),
            pl.BlockSpec((S, D), lambda: (0, 0)),
            pl.BlockSpec((S, D), lambda: (0, 0)),
            pl.BlockSpec((S, DR), lambda: (0, 0)),
            pl.BlockSpec((DC, S), lambda: (0, 0)),
        ],
        scratch_shapes=[
            pltpu.VMEM((N_DEV, S, half), jnp.bfloat16),
            pltpu.VMEM((N_DEV, half, D), jnp.bfloat16),
            pltpu.VMEM((N_DEV, half, D), jnp.bfloat16),
            pltpu.VMEM((N_DEV, S, half), jnp.bfloat16),
            pltpu.VMEM((N_DEV, half, D), jnp.bfloat16),
            pltpu.VMEM((N_DEV, half, D), jnp.bfloat16),
            pltpu.SemaphoreType.DMA((N_DEV - 1, 3)),
            pltpu.SemaphoreType.DMA((N_DEV - 1, 3)),
            pltpu.SemaphoreType.DMA((N_DEV - 1, 3)),
            pltpu.SemaphoreType.DMA((N_DEV - 1, 3)),
        ],
        compiler_params=pltpu.CompilerParams(collective_id=0),
    )(x2d, Wdkv, Wuk, Wuv, Wkr, Wqr)


HG = H // N_DEV
DG = HG * DH


def _attention_kernel(xbf, K, V, Kr, QrT, Wq):

    def body(xbf_ref, k_ref, v_ref, kr_ref, qrt_ref, wq_ref, out_ref):
        xb = xbf_ref[...]
        q = (jnp.dot(xb, wq_ref[...].astype(jnp.bfloat16),
                     preferred_element_type=jnp.float32)
             * SCALE).astype(jnp.bfloat16)
        s = lax.dot_general(q, k_ref[...].astype(jnp.bfloat16),
                            (((1,), (1,)), ((), ())),
                            preferred_element_type=jnp.float32)
        s += lax.dot_general(qrt_ref[...], kr_ref[...],
                             (((0,), (1,)), ((), ())),
                             preferred_element_type=jnp.float32)
        p = jnp.exp(s).astype(jnp.bfloat16)
        v_aug = jnp.concatenate(
            [v_ref[...].astype(jnp.bfloat16),
             jnp.ones((S, DH), jnp.bfloat16)], axis=1)
        ov = jnp.dot(p, v_aug, preferred_element_type=jnp.float32)
        out_ref[...] = (ov[:, :DH] / ov[:, DH:DH + 1]).astype(jnp.bfloat16)

    return pl.pallas_call(
        body,
        grid=(HG,),
        in_specs=[
            pl.BlockSpec((S, D), lambda h: (0, 0)),
            pl.BlockSpec((S, DH), lambda h: (0, h)),
            pl.BlockSpec((S, DH), lambda h: (0, h)),
            pl.BlockSpec((S, DR), lambda h: (0, 0)),
            pl.BlockSpec((DR, S), lambda h: (h, 0)),
            pl.BlockSpec((D, DH), lambda h: (0, h)),
        ],
        out_specs=pl.BlockSpec((S, DH), lambda h: (0, h)),
        out_shape=jax.ShapeDtypeStruct((S, DG), jnp.bfloat16),
        compiler_params=pltpu.CompilerParams(
            dimension_semantics=("arbitrary",)),
    )(xbf, K, V, Kr, QrT, Wq)


def _ring_out_proj(Og, Wo):
    hf = DG // 2

    def body(og_ref, wo_ref, out_ref,
             obufR, obufL, send_semsR, recv_semsR, send_semsL, recv_semsL):
        my = lax.axis_index("i")
        left = lax.rem(my + N_DEV - 1, N_DEV)
        right = lax.rem(my + 1, N_DEV)

        bar = pltpu.get_barrier_semaphore()
        pl.semaphore_signal(bar, inc=1, device_id=(left,),
                            device_id_type=pl.DeviceIdType.MESH)
        pl.semaphore_signal(bar, inc=1, device_id=(right,),
                            device_id_type=pl.DeviceIdType.MESH)
        pl.semaphore_wait(bar, 2)

        og = og_ref[...]
        obufR[0] = og[:, :hf]
        obufL[0] = og[:, hf:]

        def start_hop(h):
            rdmas = []
            for buf, ssems, rsems, tgt in (
                (obufR, send_semsR, recv_semsR, right),
                (obufL, send_semsL, recv_semsL, left),
            ):
                rdma = pltpu.make_async_remote_copy(
                    src_ref=buf.at[h],
                    dst_ref=buf.at[h + 1],
                    send_sem=ssems.at[h],
                    recv_sem=rsems.at[h],
                    device_id=(tgt,),
                    device_id_type=pl.DeviceIdType.MESH,
                )
                rdma.start()
                rdmas.append(rdma)
            return rdmas

        hops = [start_hop(0)]

        def wo_slice(row0, rows):
            return wo_ref[pl.ds(row0, rows), :].astype(jnp.bfloat16)

        out_ref[...] = jnp.dot(og, wo_slice(my * DG, DG),
                               preferred_element_type=jnp.float32)

        for h in range(N_DEV - 1):
            for rdma in hops[h]:
                rdma.wait_recv()
            if h < N_DEV - 2:
                hops.append(start_hop(h + 1))
            gR = lax.rem(my + N_DEV - 1 - h, N_DEV)
            gL = lax.rem(my + 1 + h, N_DEV)
            out_ref[...] += jnp.dot(obufR[h + 1], wo_slice(gR * DG, hf),
                                    preferred_element_type=jnp.float32)
            out_ref[...] += jnp.dot(obufL[h + 1], wo_slice(gL * DG + hf, hf),
                                    preferred_element_type=jnp.float32)

        for hop in hops:
            for rdma in hop:
                rdma.wait_send()

    return pl.pallas_call(
        body,
        in_specs=[
            pl.BlockSpec((S, DG), lambda: (0, 0)),
            pl.BlockSpec((D, D), lambda: (0, 0)),
        ],
        out_specs=pl.BlockSpec((S, D), lambda: (0, 0)),
        out_shape=jax.ShapeDtypeStruct((S, D), jnp.float32),
        scratch_shapes=[
            pltpu.VMEM((N_DEV, S, hf), jnp.bfloat16),
            pltpu.VMEM((N_DEV, S, hf), jnp.bfloat16),
            pltpu.SemaphoreType.DMA((N_DEV - 1,)),
            pltpu.SemaphoreType.DMA((N_DEV - 1,)),
            pltpu.SemaphoreType.DMA((N_DEV - 1,)),
            pltpu.SemaphoreType.DMA((N_DEV - 1,)),
        ],
        compiler_params=pltpu.CompilerParams(collective_id=1),
    )(Og, Wo)


def kernel(x, Wdkv, Wuk, Wuv, Wq, Wqr, Wkr, Wo):
    x2d = x.reshape(S, D)
    my = lax.axis_index("i")
    xbf, K, V, Kr, QrT = _ring_kv_kernel(x2d, Wdkv, Wuk, Wuv, Wkr, Wqr)
    Kg = lax.dynamic_slice(K, (0, my * DG), (S, DG))
    Vg = lax.dynamic_slice(V, (0, my * DG), (S, DG))
    Wqg = lax.dynamic_slice(Wq, (0, my * DG), (D, DG))
    QrTg = lax.dynamic_slice(QrT, (my * HG * DR, 0), (HG * DR, S))
    Og = _attention_kernel(xbf, Kg, Vg, Kr, QrTg, Wqg)
    out = _ring_out_proj(Og, Wo)
    return out.reshape(B, S, D)


# device time: 85922 ns/iter; 1.6156x vs baseline; 1.3830x over previous
import jax
import jax.numpy as jnp
from jax import lax
from jax.experimental import pallas as pl
from jax.experimental.pallas import tpu as pltpu

N_DEV = 4
B, S, D = 1, 1024, 2048
DC = 512
H, DH, DR = 16, 128, 32
SCALE = (DH + DR) ** -0.5


HG = H // N_DEV
DG = HG * DH


def _ring_kv_kernel(x2d, Wdkv, Wuk, Wuv, Wkr, Wqr):
    dc_sh = Wdkv.shape[1]

    half = Wdkv.shape[1] // 2

    def body(x_ref, wdkv_ref, wuk_ref, wuv_ref, wkr_ref, wqr_ref,
             xbf_ref, k_ref, v_ref, kr_ref, qrt_ref,
             cbufR, cbufL,
             wksend, wvsend, wkown, wvown, wkrecv, wvrecv,
             csend_semsR, crecv_semsR, csend_semsL, crecv_semsL,
             wsend_sems, wrecv_sems):
        my = lax.axis_index("i")
        left = lax.rem(my + N_DEV - 1, N_DEV)
        right = lax.rem(my + 1, N_DEV)

        bar = pltpu.get_barrier_semaphore()
        for g in range(N_DEV):
            @pl.when(my != g)
            def _(g=g):
                pl.semaphore_signal(bar, inc=1, device_id=(g,),
                                    device_id_type=pl.DeviceIdType.MESH)
        pl.semaphore_wait(bar, N_DEV - 1)

        xbf = x_ref[...].astype(jnp.bfloat16)
        xbf_ref[...] = xbf
        wukbf = wuk_ref[...].astype(jnp.bfloat16)
        wuvbf = wuv_ref[...].astype(jnp.bfloat16)
        cbf = jnp.dot(xbf, wdkv_ref[...].astype(jnp.bfloat16),
                      preferred_element_type=jnp.float32).astype(jnp.bfloat16)
        cbufR[0] = cbf[:, :half]
        cbufL[0] = cbf[:, half:]

        def start_chop(h):
            rdmas = []
            for buf, ssems, rsems, tgt in (
                (cbufR, csend_semsR, crecv_semsR, right),
                (cbufL, csend_semsL, crecv_semsL, left),
            ):
                rdma = pltpu.make_async_remote_copy(
                    src_ref=buf.at[h],
                    dst_ref=buf.at[h + 1],
                    send_sem=ssems.at[h],
                    recv_sem=rsems.at[h],
                    device_id=(tgt,),
                    device_id_type=pl.DeviceIdType.MESH,
                )
                rdma.start()
                rdmas.append(rdma)
            return rdmas

        chops = [start_chop(0)]

        for srel in range(1, N_DEV):
            for g in range(N_DEV):
                @pl.when(my == (g - srel) % N_DEV)
                def _(g=g, srel=srel):
                    wksend[srel - 1] = wukbf[:, g * DG:(g + 1) * DG]
                    wvsend[srel - 1] = wuvbf[:, g * DG:(g + 1) * DG]
        for g in range(N_DEV):
            @pl.when(my == g)
            def _(g=g):
                wkown[...] = wukbf[:, g * DG:(g + 1) * DG]
                wvown[...] = wuvbf[:, g * DG:(g + 1) * DG]

        w_rdmas = []
        for srel in range(1, N_DEV):
            dst = lax.rem(my + srel, N_DEV)
            for sbuf, rbuf, bidx in ((wksend, wkrecv, 0), (wvsend, wvrecv, 1)):
                rdma = pltpu.make_async_remote_copy(
                    src_ref=sbuf.at[srel - 1],
                    dst_ref=rbuf.at[srel - 1],
                    send_sem=wsend_sems.at[srel - 1, bidx],
                    recv_sem=wrecv_sems.at[srel - 1, bidx],
                    device_id=(dst,),
                    device_id_type=pl.DeviceIdType.MESH,
                )
                rdma.start()
                w_rdmas.append(rdma)

        k_ref[...] = jnp.dot(cbf, wkown[...], preferred_element_type=jnp.float32)
        v_ref[...] = jnp.dot(cbf, wvown[...], preferred_element_type=jnp.float32)
        kr_ref[...] = jnp.dot(
            xbf, wkr_ref[...].astype(jnp.bfloat16),
            preferred_element_type=jnp.float32).astype(jnp.bfloat16)
        qrt_ref[...] = (lax.dot_general(
            wqr_ref[...].astype(jnp.bfloat16), xbf,
            (((0,), (1,)), ((), ())),
            preferred_element_type=jnp.float32) * SCALE).astype(jnp.bfloat16)

        for rdma in w_rdmas:
            rdma.wait_recv()

        for h in range(N_DEV - 1):
            for rdma in chops[h]:
                rdma.wait_recv()
            if h < N_DEV - 2:
                chops.append(start_chop(h + 1))
            k_ref[...] += jnp.dot(cbufR[h + 1], wkrecv[h][:half, :],
                                  preferred_element_type=jnp.float32)
            k_ref[...] += jnp.dot(cbufL[h + 1], wkrecv[2 - h][half:, :],
                                  preferred_element_type=jnp.float32)
            v_ref[...] += jnp.dot(cbufR[h + 1], wvrecv[h][:half, :],
                                  preferred_element_type=jnp.float32)
            v_ref[...] += jnp.dot(cbufL[h + 1], wvrecv[2 - h][half:, :],
                                  preferred_element_type=jnp.float32)

        for hop in chops:
            for rdma in hop:
                rdma.wait_send()
        for rdma in w_rdmas:
            rdma.wait_send()

    return pl.pallas_call(
        body,
        out_shape=[
            jax.ShapeDtypeStruct((S, D), jnp.bfloat16),
            jax.ShapeDtypeStruct((S, DG), jnp.float32),
            jax.ShapeDtypeStruct((S, DG), jnp.float32),
            jax.ShapeDtypeStruct((S, DR), jnp.bfloat16),
            jax.ShapeDtypeStruct((DC, S), jnp.bfloat16),
        ],
        in_specs=[
            pl.BlockSpec((S, D), lambda: (0, 0)),
            pl.BlockSpec((D, dc_sh), lambda: (0, 0)),
            pl.BlockSpec((dc_sh, D), lambda: (0, 0)),
            pl.BlockSpec((dc_sh, D), lambda: (0, 0)),
            pl.BlockSpec((D, DR), lambda: (0, 0)),
            pl.BlockSpec((D, DC), lambda: (0, 0)),
        ],
        out_specs=[
            pl.BlockSpec((S, D), lambda: (0, 0)),
            pl.BlockSpec((S, DG), lambda: (0, 0)),
            pl.BlockSpec((S, DG), lambda: (0, 0)),
            pl.BlockSpec((S, DR), lambda: (0, 0)),
            pl.BlockSpec((DC, S), lambda: (0, 0)),
        ],
        scratch_shapes=[
            pltpu.VMEM((N_DEV, S, half), jnp.bfloat16),
            pltpu.VMEM((N_DEV, S, half), jnp.bfloat16),
            pltpu.VMEM((N_DEV - 1, dc_sh, DG), jnp.bfloat16),
            pltpu.VMEM((N_DEV - 1, dc_sh, DG), jnp.bfloat16),
            pltpu.VMEM((dc_sh, DG), jnp.bfloat16),
            pltpu.VMEM((dc_sh, DG), jnp.bfloat16),
            pltpu.VMEM((N_DEV - 1, dc_sh, DG), jnp.bfloat16),
            pltpu.VMEM((N_DEV - 1, dc_sh, DG), jnp.bfloat16),
            pltpu.SemaphoreType.DMA((N_DEV - 1,)),
            pltpu.SemaphoreType.DMA((N_DEV - 1,)),
            pltpu.SemaphoreType.DMA((N_DEV - 1,)),
            pltpu.SemaphoreType.DMA((N_DEV - 1,)),
            pltpu.SemaphoreType.DMA((N_DEV - 1, 2)),
            pltpu.SemaphoreType.DMA((N_DEV - 1, 2)),
        ],
        compiler_params=pltpu.CompilerParams(collective_id=0),
    )(x2d, Wdkv, Wuk, Wuv, Wkr, Wqr)


def _attention_kernel(xbf, K, V, Kr, QrT, Wq):

    def body(xbf_ref, k_ref, v_ref, kr_ref, qrt_ref, wq_ref, out_ref):
        xb = xbf_ref[...]
        q = (jnp.dot(xb, wq_ref[...].astype(jnp.bfloat16),
                     preferred_element_type=jnp.float32)
             * SCALE).astype(jnp.bfloat16)
        s = lax.dot_general(q, k_ref[...].astype(jnp.bfloat16),
                            (((1,), (1,)), ((), ())),
                            preferred_element_type=jnp.float32)
        s += lax.dot_general(qrt_ref[...], kr_ref[...],
                             (((0,), (1,)), ((), ())),
                             preferred_element_type=jnp.float32)
        p = jnp.exp(s).astype(jnp.bfloat16)
        v_aug = jnp.concatenate(
            [v_ref[...].astype(jnp.bfloat16),
             jnp.ones((S, DH), jnp.bfloat16)], axis=1)
        ov = jnp.dot(p, v_aug, preferred_element_type=jnp.float32)
        out_ref[...] = (ov[:, :DH] / ov[:, DH:DH + 1]).astype(jnp.bfloat16)

    return pl.pallas_call(
        body,
        grid=(HG,),
        in_specs=[
            pl.BlockSpec((S, D), lambda h: (0, 0)),
            pl.BlockSpec((S, DH), lambda h: (0, h)),
            pl.BlockSpec((S, DH), lambda h: (0, h)),
            pl.BlockSpec((S, DR), lambda h: (0, 0)),
            pl.BlockSpec((DR, S), lambda h: (h, 0)),
            pl.BlockSpec((D, DH), lambda h: (0, h)),
        ],
        out_specs=pl.BlockSpec((S, DH), lambda h: (0, h)),
        out_shape=jax.ShapeDtypeStruct((S, DG), jnp.bfloat16),
        compiler_params=pltpu.CompilerParams(
            dimension_semantics=("arbitrary",)),
    )(xbf, K, V, Kr, QrT, Wq)


def _ring_out_proj(Og, Wo):
    hf = DG // 2

    def body(og_ref, wo_ref, out_ref,
             obufR, obufL, send_semsR, recv_semsR, send_semsL, recv_semsL):
        my = lax.axis_index("i")
        left = lax.rem(my + N_DEV - 1, N_DEV)
        right = lax.rem(my + 1, N_DEV)

        bar = pltpu.get_barrier_semaphore()
        pl.semaphore_signal(bar, inc=1, device_id=(left,),
                            device_id_type=pl.DeviceIdType.MESH)
        pl.semaphore_signal(bar, inc=1, device_id=(right,),
                            device_id_type=pl.DeviceIdType.MESH)
        pl.semaphore_wait(bar, 2)

        og = og_ref[...]
        obufR[0] = og[:, :hf]
        obufL[0] = og[:, hf:]

        def start_hop(h):
            rdmas = []
            for buf, ssems, rsems, tgt in (
                (obufR, send_semsR, recv_semsR, right),
                (obufL, send_semsL, recv_semsL, left),
            ):
                rdma = pltpu.make_async_remote_copy(
                    src_ref=buf.at[h],
                    dst_ref=buf.at[h + 1],
                    send_sem=ssems.at[h],
                    recv_sem=rsems.at[h],
                    device_id=(tgt,),
                    device_id_type=pl.DeviceIdType.MESH,
                )
                rdma.start()
                rdmas.append(rdma)
            return rdmas

        hops = [start_hop(0)]

        def wo_slice(row0, rows):
            return wo_ref[pl.ds(row0, rows), :].astype(jnp.bfloat16)

        out_ref[...] = jnp.dot(og, wo_slice(my * DG, DG),
                               preferred_element_type=jnp.float32)

        for h in range(N_DEV - 1):
            for rdma in hops[h]:
                rdma.wait_recv()
            if h < N_DEV - 2:
                hops.append(start_hop(h + 1))
            gR = lax.rem(my + N_DEV - 1 - h, N_DEV)
            gL = lax.rem(my + 1 + h, N_DEV)
            out_ref[...] += jnp.dot(obufR[h + 1], wo_slice(gR * DG, hf),
                                    preferred_element_type=jnp.float32)
            out_ref[...] += jnp.dot(obufL[h + 1], wo_slice(gL * DG + hf, hf),
                                    preferred_element_type=jnp.float32)

        for hop in hops:
            for rdma in hop:
                rdma.wait_send()

    return pl.pallas_call(
        body,
        in_specs=[
            pl.BlockSpec((S, DG), lambda: (0, 0)),
            pl.BlockSpec((D, D), lambda: (0, 0)),
        ],
        out_specs=pl.BlockSpec((S, D), lambda: (0, 0)),
        out_shape=jax.ShapeDtypeStruct((S, D), jnp.float32),
        scratch_shapes=[
            pltpu.VMEM((N_DEV, S, hf), jnp.bfloat16),
            pltpu.VMEM((N_DEV, S, hf), jnp.bfloat16),
            pltpu.SemaphoreType.DMA((N_DEV - 1,)),
            pltpu.SemaphoreType.DMA((N_DEV - 1,)),
            pltpu.SemaphoreType.DMA((N_DEV - 1,)),
            pltpu.SemaphoreType.DMA((N_DEV - 1,)),
        ],
        compiler_params=pltpu.CompilerParams(collective_id=1),
    )(Og, Wo)


def kernel(x, Wdkv, Wuk, Wuv, Wq, Wqr, Wkr, Wo):
    x2d = x.reshape(S, D)
    my = lax.axis_index("i")
    xbf, Kg, Vg, Kr, QrT = _ring_kv_kernel(x2d, Wdkv, Wuk, Wuv, Wkr, Wqr)
    Wqg = lax.dynamic_slice(Wq, (0, my * DG), (D, DG))
    QrTg = lax.dynamic_slice(QrT, (my * HG * DR, 0), (HG * DR, S))
    Og = _attention_kernel(xbf, Kg, Vg, Kr, QrTg, Wqg)
    out = _ring_out_proj(Og, Wo)
    return out.reshape(B, S, D)


# device time: 82754 ns/iter; 1.6774x vs baseline; 1.0383x over previous
import jax
import jax.numpy as jnp
from jax import lax
from jax.experimental import pallas as pl
from jax.experimental.pallas import tpu as pltpu

N_DEV = 4
B, S, D = 1, 1024, 2048
DC = 512
H, DH, DR = 16, 128, 32
SCALE = (DH + DR) ** -0.5


HG = H // N_DEV
DG = HG * DH


def _ring_kv_kernel(x2d, Wdkv, Wuk, Wuv, Wkr, Wqr):
    dc_sh = Wdkv.shape[1]

    half = Wdkv.shape[1] // 2

    def body(x_ref, wdkv_ref, wuk_ref, wuv_ref, wkr_ref, wqr_ref,
             xbf_ref, k_ref, v_ref, kr_ref, qrt_ref,
             cbufR, cbufL,
             wksend, wvsend, wkown, wvown, wkrecv, wvrecv,
             csend_semsR, crecv_semsR, csend_semsL, crecv_semsL,
             wsend_sems, wrecv_sems):
        my = lax.axis_index("i")
        left = lax.rem(my + N_DEV - 1, N_DEV)
        right = lax.rem(my + 1, N_DEV)

        bar = pltpu.get_barrier_semaphore()
        for g in range(N_DEV):
            @pl.when(my != g)
            def _(g=g):
                pl.semaphore_signal(bar, inc=1, device_id=(g,),
                                    device_id_type=pl.DeviceIdType.MESH)
        pl.semaphore_wait(bar, N_DEV - 1)

        xbf = x_ref[...].astype(jnp.bfloat16)
        xbf_ref[...] = xbf
        wukbf = wuk_ref[...].astype(jnp.bfloat16)
        wuvbf = wuv_ref[...].astype(jnp.bfloat16)
        cbf = jnp.dot(xbf, wdkv_ref[...].astype(jnp.bfloat16),
                      preferred_element_type=jnp.float32).astype(jnp.bfloat16)
        cbufR[0] = cbf[:, :half]
        cbufL[0] = cbf[:, half:]

        def start_chop(h):
            rdmas = []
            for buf, ssems, rsems, tgt in (
                (cbufR, csend_semsR, crecv_semsR, right),
                (cbufL, csend_semsL, crecv_semsL, left),
            ):
                rdma = pltpu.make_async_remote_copy(
                    src_ref=buf.at[h],
                    dst_ref=buf.at[h + 1],
                    send_sem=ssems.at[h],
                    recv_sem=rsems.at[h],
                    device_id=(tgt,),
                    device_id_type=pl.DeviceIdType.MESH,
                )
                rdma.start()
                rdmas.append(rdma)
            return rdmas

        chops = [start_chop(0)]

        for srel in range(1, N_DEV):
            for g in range(N_DEV):
                @pl.when(my == (g - srel) % N_DEV)
                def _(g=g, srel=srel):
                    wksend[srel - 1] = wukbf[:, g * DG:(g + 1) * DG]
                    wvsend[srel - 1] = wuvbf[:, g * DG:(g + 1) * DG]
        for g in range(N_DEV):
            @pl.when(my == g)
            def _(g=g):
                wkown[...] = wukbf[:, g * DG:(g + 1) * DG]
                wvown[...] = wuvbf[:, g * DG:(g + 1) * DG]

        w_rdmas = []
        for srel in range(1, N_DEV):
            dst = lax.rem(my + srel, N_DEV)
            for sbuf, rbuf, bidx in ((wksend, wkrecv, 0), (wvsend, wvrecv, 1)):
                rdma = pltpu.make_async_remote_copy(
                    src_ref=sbuf.at[srel - 1],
                    dst_ref=rbuf.at[srel - 1],
                    send_sem=wsend_sems.at[srel - 1, bidx],
                    recv_sem=wrecv_sems.at[srel - 1, bidx],
                    device_id=(dst,),
                    device_id_type=pl.DeviceIdType.MESH,
                )
                rdma.start()
                w_rdmas.append(rdma)

        k_ref[...] = jnp.dot(cbf, wkown[...], preferred_element_type=jnp.float32)
        v_ref[...] = jnp.dot(cbf, wvown[...], preferred_element_type=jnp.float32)
        kr_ref[...] = jnp.dot(
            xbf, wkr_ref[...].astype(jnp.bfloat16),
            preferred_element_type=jnp.float32).astype(jnp.bfloat16)
        qrt_ref[...] = (lax.dot_general(
            wqr_ref[...].astype(jnp.bfloat16), xbf,
            (((0,), (1,)), ((), ())),
            preferred_element_type=jnp.float32) * SCALE).astype(jnp.bfloat16)

        for rdma in w_rdmas:
            rdma.wait_recv()

        for h in range(N_DEV - 1):
            for rdma in chops[h]:
                rdma.wait_recv()
            if h < N_DEV - 2:
                chops.append(start_chop(h + 1))
            k_ref[...] += jnp.dot(cbufR[h + 1], wkrecv[h][:half, :],
                                  preferred_element_type=jnp.float32)
            k_ref[...] += jnp.dot(cbufL[h + 1], wkrecv[2 - h][half:, :],
                                  preferred_element_type=jnp.float32)
            v_ref[...] += jnp.dot(cbufR[h + 1], wvrecv[h][:half, :],
                                  preferred_element_type=jnp.float32)
            v_ref[...] += jnp.dot(cbufL[h + 1], wvrecv[2 - h][half:, :],
                                  preferred_element_type=jnp.float32)

        for hop in chops:
            for rdma in hop:
                rdma.wait_send()
        for rdma in w_rdmas:
            rdma.wait_send()

    return pl.pallas_call(
        body,
        out_shape=[
            jax.ShapeDtypeStruct((S, D), jnp.bfloat16),
            jax.ShapeDtypeStruct((S, DG), jnp.float32),
            jax.ShapeDtypeStruct((S, DG), jnp.float32),
            jax.ShapeDtypeStruct((S, DR), jnp.bfloat16),
            jax.ShapeDtypeStruct((DC, S), jnp.bfloat16),
        ],
        in_specs=[
            pl.BlockSpec((S, D), lambda: (0, 0)),
            pl.BlockSpec((D, dc_sh), lambda: (0, 0)),
            pl.BlockSpec((dc_sh, D), lambda: (0, 0)),
            pl.BlockSpec((dc_sh, D), lambda: (0, 0)),
            pl.BlockSpec((D, DR), lambda: (0, 0)),
            pl.BlockSpec((D, DC), lambda: (0, 0)),
        ],
        out_specs=[
            pl.BlockSpec((S, D), lambda: (0, 0)),
            pl.BlockSpec((S, DG), lambda: (0, 0)),
            pl.BlockSpec((S, DG), lambda: (0, 0)),
            pl.BlockSpec((S, DR), lambda: (0, 0)),
            pl.BlockSpec((DC, S), lambda: (0, 0)),
        ],
        scratch_shapes=[
            pltpu.VMEM((N_DEV, S, half), jnp.bfloat16),
            pltpu.VMEM((N_DEV, S, half), jnp.bfloat16),
            pltpu.VMEM((N_DEV - 1, dc_sh, DG), jnp.bfloat16),
            pltpu.VMEM((N_DEV - 1, dc_sh, DG), jnp.bfloat16),
            pltpu.VMEM((dc_sh, DG), jnp.bfloat16),
            pltpu.VMEM((dc_sh, DG), jnp.bfloat16),
            pltpu.VMEM((N_DEV - 1, dc_sh, DG), jnp.bfloat16),
            pltpu.VMEM((N_DEV - 1, dc_sh, DG), jnp.bfloat16),
            pltpu.SemaphoreType.DMA((N_DEV - 1,)),
            pltpu.SemaphoreType.DMA((N_DEV - 1,)),
            pltpu.SemaphoreType.DMA((N_DEV - 1,)),
            pltpu.SemaphoreType.DMA((N_DEV - 1,)),
            pltpu.SemaphoreType.DMA((N_DEV - 1, 2)),
            pltpu.SemaphoreType.DMA((N_DEV - 1, 2)),
        ],
        compiler_params=pltpu.CompilerParams(collective_id=0),
    )(x2d, Wdkv, Wuk, Wuv, Wkr, Wqr)


def _attention_kernel(xbf, K, V, Kr, QrT, Wq):

    def body(xbf_ref, k_ref, v_ref, kr_ref, qrt_ref, wq_ref, out_ref):
        xb = xbf_ref[...]
        q = (jnp.dot(xb, wq_ref[...].astype(jnp.bfloat16),
                     preferred_element_type=jnp.float32)
             * SCALE).astype(jnp.bfloat16)
        s = lax.dot_general(q, k_ref[...].astype(jnp.bfloat16),
                            (((1,), (1,)), ((), ())),
                            preferred_element_type=jnp.float32)
        s += lax.dot_general(qrt_ref[...], kr_ref[...],
                             (((0,), (1,)), ((), ())),
                             preferred_element_type=jnp.float32)
        p = jnp.exp(s).astype(jnp.bfloat16)
        v_aug = jnp.concatenate(
            [v_ref[...].astype(jnp.bfloat16),
             jnp.ones((S, DH), jnp.bfloat16)], axis=1)
        ov = jnp.dot(p, v_aug, preferred_element_type=jnp.float32)
        out_ref[...] = (ov[:, :DH] / ov[:, DH:DH + 1]).astype(jnp.bfloat16)

    return pl.pallas_call(
        body,
        grid=(HG,),
        in_specs=[
            pl.BlockSpec((S, D), lambda h: (0, 0)),
            pl.BlockSpec((S, DH), lambda h: (0, h)),
            pl.BlockSpec((S, DH), lambda h: (0, h)),
            pl.BlockSpec((S, DR), lambda h: (0, 0)),
            pl.BlockSpec((DR, S), lambda h: (h, 0)),
            pl.BlockSpec((D, DH), lambda h: (0, h)),
        ],
        out_specs=pl.BlockSpec((S, DH), lambda h: (0, h)),
        out_shape=jax.ShapeDtypeStruct((S, DG), jnp.bfloat16),
        compiler_params=pltpu.CompilerParams(
            dimension_semantics=("arbitrary",)),
    )(xbf, K, V, Kr, QrT, Wq)


def _attn_out_fused(xbf, Kg, Vg, Kr, QrTg, Wqg, Wo):

    def body(xbf_ref, k_ref, v_ref, kr_ref, qrt_ref, wqg_ref, wo_ref,
             out_ref, oown, orecv, send_sems, recv_sems):
        my = lax.axis_index("i")

        bar = pltpu.get_barrier_semaphore()
        for g in range(N_DEV):
            @pl.when(my != g)
            def _(g=g):
                pl.semaphore_signal(bar, inc=1, device_id=(g,),
                                    device_id_type=pl.DeviceIdType.MESH)
        pl.semaphore_wait(bar, N_DEV - 1)

        xb = xbf_ref[...]
        rdmas = []
        for h in range(HG):
            sl = slice(h * DH, (h + 1) * DH)
            rsl = slice(h * DR, (h + 1) * DR)
            q = (jnp.dot(xb, wqg_ref[:, sl].astype(jnp.bfloat16),
                         preferred_element_type=jnp.float32)
                 * SCALE).astype(jnp.bfloat16)
            s = lax.dot_general(q, k_ref[:, sl].astype(jnp.bfloat16),
                                (((1,), (1,)), ((), ())),
                                preferred_element_type=jnp.float32)
            s += lax.dot_general(qrt_ref[rsl, :], kr_ref[...],
                                 (((0,), (1,)), ((), ())),
                                 preferred_element_type=jnp.float32)
            p = jnp.exp(s).astype(jnp.bfloat16)
            v_aug = jnp.concatenate(
                [v_ref[:, sl].astype(jnp.bfloat16),
                 jnp.ones((S, DH), jnp.bfloat16)], axis=1)
            ov = jnp.dot(p, v_aug, preferred_element_type=jnp.float32)
            oown[:, sl] = (ov[:, :DH] / ov[:, DH:DH + 1]).astype(jnp.bfloat16)
            for srel in range(1, N_DEV):
                dst = lax.rem(my + srel, N_DEV)
                rdma = pltpu.make_async_remote_copy(
                    src_ref=oown.at[:, sl],
                    dst_ref=orecv.at[N_DEV - 1 - srel, :, sl],
                    send_sem=send_sems.at[h, srel - 1],
                    recv_sem=recv_sems.at[h, srel - 1],
                    device_id=(dst,),
                    device_id_type=pl.DeviceIdType.MESH,
                )
                rdma.start()
                rdmas.append(rdma)

        def wo_slice(row0):
            return wo_ref[pl.ds(row0, DG), :].astype(jnp.bfloat16)

        out_ref[...] = jnp.dot(oown[...], wo_slice(my * DG),
                               preferred_element_type=jnp.float32)

        for rdma in rdmas:
            rdma.wait_recv()
        for t in range(N_DEV - 1):
            origin = lax.rem(my + t + 1, N_DEV)
            out_ref[...] += jnp.dot(orecv[t], wo_slice(origin * DG),
                                    preferred_element_type=jnp.float32)

        for rdma in rdmas:
            rdma.wait_send()

    return pl.pallas_call(
        body,
        in_specs=[
            pl.BlockSpec((S, D), lambda: (0, 0)),
            pl.BlockSpec((S, DG), lambda: (0, 0)),
            pl.BlockSpec((S, DG), lambda: (0, 0)),
            pl.BlockSpec((S, DR), lambda: (0, 0)),
            pl.BlockSpec((HG * DR, S), lambda: (0, 0)),
            pl.BlockSpec((D, DG), lambda: (0, 0)),
            pl.BlockSpec((D, D), lambda: (0, 0)),
        ],
        out_specs=pl.BlockSpec((S, D), lambda: (0, 0)),
        out_shape=jax.ShapeDtypeStruct((S, D), jnp.float32),
        scratch_shapes=[
            pltpu.VMEM((S, DG), jnp.bfloat16),
            pltpu.VMEM((N_DEV - 1, S, DG), jnp.bfloat16),
            pltpu.SemaphoreType.DMA((HG, N_DEV - 1)),
            pltpu.SemaphoreType.DMA((HG, N_DEV - 1)),
        ],
        compiler_params=pltpu.CompilerParams(collective_id=1),
    )(xbf, Kg, Vg, Kr, QrTg, Wqg, Wo)


def kernel(x, Wdkv, Wuk, Wuv, Wq, Wqr, Wkr, Wo):
    x2d = x.reshape(S, D)
    my = lax.axis_index("i")
    xbf, Kg, Vg, Kr, QrT = _ring_kv_kernel(x2d, Wdkv, Wuk, Wuv, Wkr, Wqr)
    Wqg = lax.dynamic_slice(Wq, (0, my * DG), (D, DG))
    QrTg = lax.dynamic_slice(QrT, (my * HG * DR, 0), (HG * DR, S))
    out = _attn_out_fused(xbf, Kg, Vg, Kr, QrTg, Wqg, Wo)
    return out.reshape(B, S, D)


# device time: 82390 ns/iter; 1.6848x vs baseline; 1.0044x over previous
import jax
import jax.numpy as jnp
from jax import lax
from jax.experimental import pallas as pl
from jax.experimental.pallas import tpu as pltpu

N_DEV = 4
B, S, D = 1, 1024, 2048
DC = 512
H, DH, DR = 16, 128, 32
SCALE = (DH + DR) ** -0.5


HG = H // N_DEV
DG = HG * DH


def _ring_kv_kernel(x2d, Wdkv, Wuk, Wuv, Wkr, Wqr):
    dc_sh = Wdkv.shape[1]

    half = Wdkv.shape[1] // 2

    def body(x_ref, wdkv_ref, wuk_ref, wuv_ref, wkr_ref, wqr_ref,
             xbf_ref, k_ref, v_ref, kr_ref, qrt_ref,
             cbufR, cbufL,
             wksend, wvsend, wkown, wvown, wkrecv, wvrecv,
             csend_semsR, crecv_semsR, csend_semsL, crecv_semsL,
             wsend_sems, wrecv_sems):
        my = lax.axis_index("i")
        left = lax.rem(my + N_DEV - 1, N_DEV)
        right = lax.rem(my + 1, N_DEV)

        bar = pltpu.get_barrier_semaphore()
        for g in range(N_DEV):
            @pl.when(my != g)
            def _(g=g):
                pl.semaphore_signal(bar, inc=1, device_id=(g,),
                                    device_id_type=pl.DeviceIdType.MESH)
        pl.semaphore_wait(bar, N_DEV - 1)

        xbf = x_ref[...].astype(jnp.bfloat16)
        xbf_ref[...] = xbf
        wukbf = wuk_ref[...].astype(jnp.bfloat16)
        wuvbf = wuv_ref[...].astype(jnp.bfloat16)
        cbf = jnp.dot(xbf, wdkv_ref[...].astype(jnp.bfloat16),
                      preferred_element_type=jnp.float32).astype(jnp.bfloat16)
        cbufR[0] = cbf[:, :half]
        cbufL[0] = cbf[:, half:]

        def start_chop(h):
            rdmas = []
            for buf, ssems, rsems, tgt in (
                (cbufR, csend_semsR, crecv_semsR, right),
                (cbufL, csend_semsL, crecv_semsL, left),
            ):
                rdma = pltpu.make_async_remote_copy(
                    src_ref=buf.at[h],
                    dst_ref=buf.at[h + 1],
                    send_sem=ssems.at[h],
                    recv_sem=rsems.at[h],
                    device_id=(tgt,),
                    device_id_type=pl.DeviceIdType.MESH,
                )
                rdma.start()
                rdmas.append(rdma)
            return rdmas

        chops = [start_chop(0)]

        for srel in range(1, N_DEV):
            for g in range(N_DEV):
                @pl.when(my == (g - srel) % N_DEV)
                def _(g=g, srel=srel):
                    wksend[srel - 1] = wukbf[:, g * DG:(g + 1) * DG]
                    wvsend[srel - 1] = wuvbf[:, g * DG:(g + 1) * DG]
        for g in range(N_DEV):
            @pl.when(my == g)
            def _(g=g):
                wkown[...] = wukbf[:, g * DG:(g + 1) * DG]
                wvown[...] = wuvbf[:, g * DG:(g + 1) * DG]

        w_rdmas = []
        for srel in range(1, N_DEV):
            dst = lax.rem(my + srel, N_DEV)
            for sbuf, rbuf, bidx in ((wksend, wkrecv, 0), (wvsend, wvrecv, 1)):
                rdma = pltpu.make_async_remote_copy(
                    src_ref=sbuf.at[srel - 1],
                    dst_ref=rbuf.at[srel - 1],
                    send_sem=wsend_sems.at[srel - 1, bidx],
                    recv_sem=wrecv_sems.at[srel - 1, bidx],
                    device_id=(dst,),
                    device_id_type=pl.DeviceIdType.MESH,
                )
                rdma.start()
                w_rdmas.append(rdma)

        k_ref[...] = jnp.dot(cbf, wkown[...], preferred_element_type=jnp.float32)
        v_ref[...] = jnp.dot(cbf, wvown[...], preferred_element_type=jnp.float32)
        kr_ref[...] = jnp.dot(
            xbf, wkr_ref[...].astype(jnp.bfloat16),
            preferred_element_type=jnp.float32).astype(jnp.bfloat16)
        qrt_ref[...] = (lax.dot_general(
            wqr_ref[...].astype(jnp.bfloat16), xbf,
            (((0,), (1,)), ((), ())),
            preferred_element_type=jnp.float32) * SCALE).astype(jnp.bfloat16)

        for srel in (1, 3):
            w_rdmas[(srel - 1) * 2].wait_recv()
            w_rdmas[(srel - 1) * 2 + 1].wait_recv()

        for h in range(N_DEV - 1):
            if h == 1:
                w_rdmas[2].wait_recv()
                w_rdmas[3].wait_recv()
            for rdma in chops[h]:
                rdma.wait_recv()
            if h < N_DEV - 2:
                chops.append(start_chop(h + 1))
            k_ref[...] += jnp.dot(cbufR[h + 1], wkrecv[h][:half, :],
                                  preferred_element_type=jnp.float32)
            k_ref[...] += jnp.dot(cbufL[h + 1], wkrecv[2 - h][half:, :],
                                  preferred_element_type=jnp.float32)
            v_ref[...] += jnp.dot(cbufR[h + 1], wvrecv[h][:half, :],
                                  preferred_element_type=jnp.float32)
            v_ref[...] += jnp.dot(cbufL[h + 1], wvrecv[2 - h][half:, :],
                                  preferred_element_type=jnp.float32)

        for hop in chops:
            for rdma in hop:
                rdma.wait_send()
        for rdma in w_rdmas:
            rdma.wait_send()

    return pl.pallas_call(
        body,
        out_shape=[
            jax.ShapeDtypeStruct((S, D), jnp.bfloat16),
            jax.ShapeDtypeStruct((S, DG), jnp.float32),
            jax.ShapeDtypeStruct((S, DG), jnp.float32),
            jax.ShapeDtypeStruct((S, DR), jnp.bfloat16),
            jax.ShapeDtypeStruct((DC, S), jnp.bfloat16),
        ],
        in_specs=[
            pl.BlockSpec((S, D), lambda: (0, 0)),
            pl.BlockSpec((D, dc_sh), lambda: (0, 0)),
            pl.BlockSpec((dc_sh, D), lambda: (0, 0)),
            pl.BlockSpec((dc_sh, D), lambda: (0, 0)),
            pl.BlockSpec((D, DR), lambda: (0, 0)),
            pl.BlockSpec((D, DC), lambda: (0, 0)),
        ],
        out_specs=[
            pl.BlockSpec((S, D), lambda: (0, 0)),
            pl.BlockSpec((S, DG), lambda: (0, 0)),
            pl.BlockSpec((S, DG), lambda: (0, 0)),
            pl.BlockSpec((S, DR), lambda: (0, 0)),
            pl.BlockSpec((DC, S), lambda: (0, 0)),
        ],
        scratch_shapes=[
            pltpu.VMEM((N_DEV, S, half), jnp.bfloat16),
            pltpu.VMEM((N_DEV, S, half), jnp.bfloat16),
            pltpu.VMEM((N_DEV - 1, dc_sh, DG), jnp.bfloat16),
            pltpu.VMEM((N_DEV - 1, dc_sh, DG), jnp.bfloat16),
            pltpu.VMEM((dc_sh, DG), jnp.bfloat16),
            pltpu.VMEM((dc_sh, DG), jnp.bfloat16),
            pltpu.VMEM((N_DEV - 1, dc_sh, DG), jnp.bfloat16),
            pltpu.VMEM((N_DEV - 1, dc_sh, DG), jnp.bfloat16),
            pltpu.SemaphoreType.DMA((N_DEV - 1,)),
            pltpu.SemaphoreType.DMA((N_DEV - 1,)),
            pltpu.SemaphoreType.DMA((N_DEV - 1,)),
            pltpu.SemaphoreType.DMA((N_DEV - 1,)),
            pltpu.SemaphoreType.DMA((N_DEV - 1, 2)),
            pltpu.SemaphoreType.DMA((N_DEV - 1, 2)),
        ],
        compiler_params=pltpu.CompilerParams(collective_id=0),
    )(x2d, Wdkv, Wuk, Wuv, Wkr, Wqr)


def _attention_kernel(xbf, K, V, Kr, QrT, Wq):

    def body(xbf_ref, k_ref, v_ref, kr_ref, qrt_ref, wq_ref, out_ref):
        xb = xbf_ref[...]
        q = (jnp.dot(xb, wq_ref[...].astype(jnp.bfloat16),
                     preferred_element_type=jnp.float32)
             * SCALE).astype(jnp.bfloat16)
        s = lax.dot_general(q, k_ref[...].astype(jnp.bfloat16),
                            (((1,), (1,)), ((), ())),
                            preferred_element_type=jnp.float32)
        s += lax.dot_general(qrt_ref[...], kr_ref[...],
                             (((0,), (1,)), ((), ())),
                             preferred_element_type=jnp.float32)
        p = jnp.exp(s).astype(jnp.bfloat16)
        v_aug = jnp.concatenate(
            [v_ref[...].astype(jnp.bfloat16),
             jnp.ones((S, DH), jnp.bfloat16)], axis=1)
        ov = jnp.dot(p, v_aug, preferred_element_type=jnp.float32)
        out_ref[...] = (ov[:, :DH] / ov[:, DH:DH + 1]).astype(jnp.bfloat16)

    return pl.pallas_call(
        body,
        grid=(HG,),
        in_specs=[
            pl.BlockSpec((S, D), lambda h: (0, 0)),
            pl.BlockSpec((S, DH), lambda h: (0, h)),
            pl.BlockSpec((S, DH), lambda h: (0, h)),
            pl.BlockSpec((S, DR), lambda h: (0, 0)),
            pl.BlockSpec((DR, S), lambda h: (h, 0)),
            pl.BlockSpec((D, DH), lambda h: (0, h)),
        ],
        out_specs=pl.BlockSpec((S, DH), lambda h: (0, h)),
        out_shape=jax.ShapeDtypeStruct((S, DG), jnp.bfloat16),
        compiler_params=pltpu.CompilerParams(
            dimension_semantics=("arbitrary",)),
    )(xbf, K, V, Kr, QrT, Wq)


def _attn_out_fused(xbf, Kg, Vg, Kr, QrTg, Wqg, Wo):

    def body(xbf_ref, k_ref, v_ref, kr_ref, qrt_ref, wqg_ref, wo_ref,
             out_ref, oown, orecv, send_sems, recv_sems):
        my = lax.axis_index("i")

        bar = pltpu.get_barrier_semaphore()
        for g in range(N_DEV):
            @pl.when(my != g)
            def _(g=g):
                pl.semaphore_signal(bar, inc=1, device_id=(g,),
                                    device_id_type=pl.DeviceIdType.MESH)
        pl.semaphore_wait(bar, N_DEV - 1)

        xb = xbf_ref[...]
        rdmas = []
        for h in range(HG):
            sl = slice(h * DH, (h + 1) * DH)
            rsl = slice(h * DR, (h + 1) * DR)
            q = (jnp.dot(xb, wqg_ref[:, sl].astype(jnp.bfloat16),
                         preferred_element_type=jnp.float32)
                 * SCALE).astype(jnp.bfloat16)
            s = lax.dot_general(q, k_ref[:, sl].astype(jnp.bfloat16),
                                (((1,), (1,)), ((), ())),
                                preferred_element_type=jnp.float32)
            s += lax.dot_general(qrt_ref[rsl, :], kr_ref[...],
                                 (((0,), (1,)), ((), ())),
                                 preferred_element_type=jnp.float32)
            p = jnp.exp(s).astype(jnp.bfloat16)
            v_aug = jnp.concatenate(
                [v_ref[:, sl].astype(jnp.bfloat16),
                 jnp.ones((S, DH), jnp.bfloat16)], axis=1)
            ov = jnp.dot(p, v_aug, preferred_element_type=jnp.float32)
            oown[:, sl] = (ov[:, :DH] / ov[:, DH:DH + 1]).astype(jnp.bfloat16)
            for srel in range(1, N_DEV):
                dst = lax.rem(my + srel, N_DEV)
                rdma = pltpu.make_async_remote_copy(
                    src_ref=oown.at[:, sl],
                    dst_ref=orecv.at[N_DEV - 1 - srel, :, sl],
                    send_sem=send_sems.at[h, srel - 1],
                    recv_sem=recv_sems.at[h, srel - 1],
                    device_id=(dst,),
                    device_id_type=pl.DeviceIdType.MESH,
                )
                rdma.start()
                rdmas.append(rdma)

        def wo_slice(row0):
            return wo_ref[pl.ds(row0, DG), :].astype(jnp.bfloat16)

        out_ref[...] = jnp.dot(oown[...], wo_slice(my * DG),
                               preferred_element_type=jnp.float32)

        for t in (0, 2, 1):
            srel = N_DEV - 1 - t
            for h in range(HG):
                rdmas[h * (N_DEV - 1) + srel - 1].wait_recv()
            origin = lax.rem(my + t + 1, N_DEV)
            out_ref[...] += jnp.dot(orecv[t], wo_slice(origin * DG),
                                    preferred_element_type=jnp.float32)

        for rdma in rdmas:
            rdma.wait_send()

    return pl.pallas_call(
        body,
        in_specs=[
            pl.BlockSpec((S, D), lambda: (0, 0)),
            pl.BlockSpec((S, DG), lambda: (0, 0)),
            pl.BlockSpec((S, DG), lambda: (0, 0)),
            pl.BlockSpec((S, DR), lambda: (0, 0)),
            pl.BlockSpec((HG * DR, S), lambda: (0, 0)),
            pl.BlockSpec((D, DG), lambda: (0, 0)),
            pl.BlockSpec((D, D), lambda: (0, 0)),
        ],
        out_specs=pl.BlockSpec((S, D), lambda: (0, 0)),
        out_shape=jax.ShapeDtypeStruct((S, D), jnp.float32),
        scratch_shapes=[
            pltpu.VMEM((S, DG), jnp.bfloat16),
            pltpu.VMEM((N_DEV - 1, S, DG), jnp.bfloat16),
            pltpu.SemaphoreType.DMA((HG, N_DEV - 1)),
            pltpu.SemaphoreType.DMA((HG, N_DEV - 1)),
        ],
        compiler_params=pltpu.CompilerParams(collective_id=1),
    )(xbf, Kg, Vg, Kr, QrTg, Wqg, Wo)


def kernel(x, Wdkv, Wuk, Wuv, Wq, Wqr, Wkr, Wo):
    x2d = x.reshape(S, D)
    my = lax.axis_index("i")
    xbf, Kg, Vg, Kr, QrT = _ring_kv_kernel(x2d, Wdkv, Wuk, Wuv, Wkr, Wqr)
    Wqg = lax.dynamic_slice(Wq, (0, my * DG), (D, DG))
    QrTg = lax.dynamic_slice(QrT, (my * HG * DR, 0), (HG * DR, S))
    out = _attn_out_fused(xbf, Kg, Vg, Kr, QrTg, Wqg, Wo)
    return out.reshape(B, S, D)
